# Initial kernel scaffold; baseline (speedup 1.0000x reference)
import jax
import jax.numpy as jnp
from jax.experimental import pallas as pl


def kernel(x_cat, x_cont, hist_seq, cat_tables, seq_table, W1, b1, W2, b2):
    raise NotImplementedError("write your pallas kernel here")


# speedup vs baseline: 1.8912x; 1.8912x over previous
import functools

import jax
import jax.numpy as jnp
from jax import lax
from jax.experimental import pallas as pl
from jax.experimental.pallas import tpu as pltpu
from jax.experimental.pallas import tpu_sc as plsc

B = 4096
NCAT = 26
VCAT = 100000
L = 50
D = 32
NCONT = 13
HID = 128

NW = 32
BPW = B // NW
CAT_PW = BPW * NCAT
SEQ_PW = BPW * L
CHUNK = 128
NCAT_CHUNKS = CAT_PW // CHUNK
NSEQ_CHUNKS = SEQ_PW // CHUNK

_sc_mesh = plsc.VectorSubcoreMesh(core_axis_name="c", subcore_axis_name="s")


@functools.partial(
    pl.kernel,
    mesh=_sc_mesh,
    compiler_params=pltpu.CompilerParams(use_tc_tiling_on_sc=False),
    out_type=[
        jax.ShapeDtypeStruct((B * NCAT, D), jnp.float32),
        jax.ShapeDtypeStruct((B, D), jnp.float32),
    ],
    scratch_types=[
        pltpu.VMEM((NCAT_CHUNKS, CHUNK), jnp.int32),
        pltpu.VMEM((NSEQ_CHUNKS, CHUNK), jnp.int32),
        pltpu.VMEM((NSEQ_CHUNKS, CHUNK), jnp.int32),
        pltpu.VMEM((CHUNK, D), jnp.float32),
        pltpu.VMEM((CHUNK, D), jnp.float32),
        pltpu.VMEM_SHARED((16 * BPW, D), jnp.float32),
        pltpu.VMEM((BPW, D), jnp.float32),
        pltpu.SemaphoreType.DMA,
        pltpu.SemaphoreType.DMA,
    ],
)
def _sc_gather(idx_cat_hbm, idx_seq_hbm, pat_hbm, zeros_hbm,
               cat_tab_hbm, seq_tab_hbm,
               cat_out_hbm, seq_out_hbm,
               idxc_v, idxs_v, pat_v, rowc_v, rows_v, acc_shared, tmp_v,
               sem_g, sem_s):
    sid = lax.axis_index("s")
    wid = sid * 2 + lax.axis_index("c")

    pltpu.sync_copy(idx_cat_hbm.at[wid], idxc_v)
    pltpu.sync_copy(idx_seq_hbm.at[wid], idxs_v)
    pltpu.sync_copy(pat_hbm.at[sid], pat_v)
    pltpu.sync_copy(zeros_hbm, tmp_v)
    pltpu.sync_copy(tmp_v, acc_shared.at[pl.ds(sid * BPW, BPW)])

    def cat_body(k, carry):
        pltpu.async_copy(cat_tab_hbm.at[idxc_v.at[k]], rowc_v, sem_g).wait()
        pltpu.sync_copy(
            rowc_v, cat_out_hbm.at[pl.ds(wid * CAT_PW + k * CHUNK, CHUNK)])
        return carry

    lax.fori_loop(0, NCAT_CHUNKS, cat_body, 0)

    def seq_body(k, carry):
        pltpu.async_copy(seq_tab_hbm.at[idxs_v.at[k]], rows_v, sem_g).wait()
        pltpu.async_copy(rows_v, acc_shared.at[pat_v.at[k]], sem_s,
                         add=True).wait()
        return carry

    lax.fori_loop(0, NSEQ_CHUNKS, seq_body, 0)

    pltpu.sync_copy(acc_shared.at[pl.ds(sid * BPW, BPW)], tmp_v)
    pltpu.sync_copy(tmp_v, seq_out_hbm.at[pl.ds(wid * BPW, BPW)])


BLK = 512
CAT_F = NCAT * D


def _mlp_body(cat_ref, cont_ref, seq_ref, w1c_ref, w1x_ref, w1s_ref,
              b1_ref, w2_ref, b2_ref, out_ref):
    h = (jnp.dot(cat_ref[...], w1c_ref[...], preferred_element_type=jnp.float32)
         + jnp.dot(cont_ref[...], w1x_ref[...], preferred_element_type=jnp.float32)
         + jnp.dot(seq_ref[...] * (1.0 / L), w1s_ref[...],
                   preferred_element_type=jnp.float32)
         + b1_ref[...])
    h = jnp.maximum(h, 0.0)
    logits = jnp.sum(h * w2_ref[...], axis=1) + b2_ref[0, 0]
    out_ref[...] = logits[None, :]


def kernel(x_cat, x_cont, hist_seq, cat_tables, seq_table, W1, b1, W2, b2):
    offs = (jnp.arange(NCAT, dtype=jnp.int32) * VCAT)[None, :]
    idx_cat = (x_cat + offs).reshape(NW, NCAT_CHUNKS, CHUNK)
    idx_seq = hist_seq.reshape(NW, NSEQ_CHUNKS, CHUNK)
    base_pat = jnp.arange(SEQ_PW, dtype=jnp.int32) // L
    pat = (base_pat[None, :] + jnp.arange(16, dtype=jnp.int32)[:, None] * BPW
           ).reshape(16, NSEQ_CHUNKS, CHUNK)
    zeros = jnp.zeros((BPW, D), jnp.float32)

    cat_rows, seq_sum = _sc_gather(
        idx_cat, idx_seq, pat, zeros,
        cat_tables.reshape(NCAT * VCAT, D), seq_table)

    cat_flat = cat_rows.reshape(B, CAT_F)

    out = pl.pallas_call(
        _mlp_body,
        grid=(B // BLK,),
        in_specs=[
            pl.BlockSpec((BLK, CAT_F), lambda i: (i, 0)),
            pl.BlockSpec((BLK, NCONT), lambda i: (i, 0)),
            pl.BlockSpec((BLK, D), lambda i: (i, 0)),
            pl.BlockSpec((CAT_F, HID), lambda i: (0, 0)),
            pl.BlockSpec((NCONT, HID), lambda i: (0, 0)),
            pl.BlockSpec((D, HID), lambda i: (0, 0)),
            pl.BlockSpec((1, HID), lambda i: (0, 0)),
            pl.BlockSpec((1, HID), lambda i: (0, 0)),
            pl.BlockSpec((1, 1), lambda i: (0, 0)),
        ],
        out_specs=pl.BlockSpec((1, BLK), lambda i: (0, i)),
        out_shape=jax.ShapeDtypeStruct((1, B), jnp.float32),
    )(cat_flat, x_cont, seq_sum,
      W1[:CAT_F], W1[CAT_F:CAT_F + NCONT], W1[CAT_F + NCONT:],
      b1[None, :], W2.T, b2[None, :])

    return out[0]
